# baseline (speedup 1.0000x reference)
import jax
import jax.numpy as jnp
from jax.experimental import pallas as pl
from jax.experimental.pallas import tpu as pltpu

_B = 64
_PLANES = 64 * 5 * 26
_NP = 130
_GRID = _PLANES // _NP // 2


def _tc_body(a_ref, b_ref, o_ref, acc_ref):
    i = pl.program_id(0)

    @pl.when(i == 0)
    def _():
        acc_ref[0, 0] = 0.0

    ca = a_ref[:, :, 4]
    cb = b_ref[:, :, 4]
    sa = 1.0 / (1.0 + jnp.exp(-ca))
    sb = 1.0 / (1.0 + jnp.exp(-cb))
    acc_ref[0, 0] += jnp.sum(sa * sa) + jnp.sum(sb * sb)

    @pl.when(i == _GRID - 1)
    def _():
        o_ref[0, 0] = acc_ref[0, 0]


_tc_call = pl.pallas_call(
    _tc_body,
    grid=(_GRID,),
    in_specs=[
        pl.BlockSpec((_NP, 26, 85), lambda i: (i, 0, 0), memory_space=pltpu.VMEM),
        pl.BlockSpec(
            (_NP, 26, 85), lambda i: (i + _GRID, 0, 0), memory_space=pltpu.VMEM
        ),
    ],
    out_specs=pl.BlockSpec((1, 1), lambda i: (0, 0), memory_space=pltpu.SMEM),
    out_shape=jax.ShapeDtypeStruct((1, 1), jnp.float32),
    scratch_shapes=[pltpu.SMEM((1, 1), jnp.float32)],
)


def kernel(predictions, targets):
    pred3 = predictions.reshape(_PLANES, 26, 85)
    out = _tc_call(pred3, pred3)
    return out[0, 0] * (1.0 / _B)
